# baseline (speedup 1.0000x reference)
import dataclasses
import functools

import jax
import jax.numpy as jnp
from jax import lax
from jax.experimental import pallas as pl
from jax.experimental.pallas import tpu as pltpu
from jax.experimental.pallas import tpu_sc as plsc

N = 10000
D = 256
E = 160000
K_ACTIVE = 1000
NSUB = 16
NW = 32
ROWS = 320
NPAD = NW * ROWS

EDGES_PER_SCAN = E // NSUB
CH = 400
NCHUNK_A = EDGES_PER_SCAN // CH
VECS = CH // 16
REGION = 10400
HALF_N = 16 * ROWS
GB = 16
STAGE = CH + 16

_i32 = jnp.int32
_f32 = jnp.float32


def _sc_body(x_hbm, src_hbm, dst_hbm, rank_hbm, z_acc, z_deg,
             s_out, deg_out,
             rank_v, chunk_s, chunk_d, chunk_s2, chunk_d2,
             st_a, st_b, st_a2, st_b2,
             cntbuf, cntv, hbuf, acc, dacc,
             sem_s0, sem_d0, sem_s1, sem_d1, sem_z0, sem_z1,
             sp_src, sp_dst, sp_cnt):
    c = lax.axis_index("c")
    s = lax.axis_index("s")
    w = c * NSUB + s
    lo = w * ROWS
    iota = lax.iota(_i32, 16)
    ones_f = jnp.ones((16,), _f32)

    pltpu.async_copy(z_acc, acc.at[pl.ds(0, ROWS * D)], sem_z0)
    pltpu.async_copy(z_deg, dacc, sem_z1)

    pltpu.sync_copy(rank_hbm, rank_v)
    base = s * EDGES_PER_SCAN

    def _flush_a0(nf):
        pltpu.sync_copy(st_a.at[pl.ds(0, CH)],
                        sp_src.at[pl.ds(s * REGION + nf * CH, CH)])
        pltpu.sync_copy(st_b.at[pl.ds(0, CH)],
                        sp_dst.at[pl.ds(s * REGION + nf * CH, CH)])

    def _flush_a1(nf):
        off = s * REGION + REGION - (nf + 1) * CH
        pltpu.sync_copy(st_a2.at[pl.ds(0, CH)],
                        sp_src.at[pl.ds(off, CH)])
        pltpu.sync_copy(st_b2.at[pl.ds(0, CH)],
                        sp_dst.at[pl.ds(off, CH)])

    def _start_a(t, cs, cd, ss, sd):
        pltpu.async_copy(src_hbm.at[pl.ds(base + t * CH, CH)], cs, ss)
        pltpu.async_copy(dst_hbm.at[pl.ds(base + t * CH, CH)], cd, sd)

    def _wait_a(t, cs, cd, ss, sd):
        pltpu.make_async_copy(src_hbm.at[pl.ds(base + t * CH, CH)],
                              cs, ss).wait()
        pltpu.make_async_copy(dst_hbm.at[pl.ds(base + t * CH, CH)],
                              cd, sd).wait()

    def _process_a(t, cs, cd, carry):
        def _vec_a(v, carry):
            cur0, nf0, cur1, nf1 = carry
            s16 = cs[pl.ds(v * 16, 16)]
            d16 = cd[pl.ds(v * 16, 16)]
            r16 = plsc.load_gather(rank_v, [s16])
            act = r16 <= K_ACTIVE
            keep0 = act & (d16 < HALF_N)
            keep1 = act & (d16 >= HALF_N)
            plsc.store_compressed(st_a.at[pl.ds(cur0, 16)], s16, mask=keep0)
            plsc.store_compressed(st_b.at[pl.ds(cur0, 16)], d16, mask=keep0)
            plsc.store_compressed(st_a2.at[pl.ds(cur1, 16)], s16, mask=keep1)
            plsc.store_compressed(st_b2.at[pl.ds(cur1, 16)], d16, mask=keep1)
            cur0 = cur0 + jnp.max(plsc.all_reduce_population_count(keep0))
            cur1 = cur1 + jnp.max(plsc.all_reduce_population_count(keep1))
            f0 = cur0 >= CH
            f1 = cur1 >= CH

            @pl.when(f0)
            def _():
                _flush_a0(nf0)
                st_a[pl.ds(0, 16)] = st_a[pl.ds(CH, 16)]
                st_b[pl.ds(0, 16)] = st_b[pl.ds(CH, 16)]

            @pl.when(f1)
            def _():
                _flush_a1(nf1)
                st_a2[pl.ds(0, 16)] = st_a2[pl.ds(CH, 16)]
                st_b2[pl.ds(0, 16)] = st_b2[pl.ds(CH, 16)]

            cur0 = jnp.where(f0, cur0 - CH, cur0)
            nf0 = jnp.where(f0, nf0 + 1, nf0)
            cur1 = jnp.where(f1, cur1 - CH, cur1)
            nf1 = jnp.where(f1, nf1 + 1, nf1)
            return cur0, nf0, cur1, nf1

        return lax.fori_loop(0, VECS, _vec_a, carry)

    _start_a(0, chunk_s, chunk_d, sem_s0, sem_d0)

    def _pair_a(i, carry):
        t0 = 2 * i
        _start_a(t0 + 1, chunk_s2, chunk_d2, sem_s1, sem_d1)
        _wait_a(t0, chunk_s, chunk_d, sem_s0, sem_d0)
        carry = _process_a(t0, chunk_s, chunk_d, carry)
        _start_a(t0 + 2, chunk_s, chunk_d, sem_s0, sem_d0)
        _wait_a(t0 + 1, chunk_s2, chunk_d2, sem_s1, sem_d1)
        return _process_a(t0 + 1, chunk_s2, chunk_d2, carry)

    zero4 = (jnp.int32(0),) * 4
    carry = lax.fori_loop(0, (NCHUNK_A - 1) // 2, _pair_a, zero4)
    _wait_a(NCHUNK_A - 1, chunk_s, chunk_d, sem_s0, sem_d0)
    cur0, nf0, cur1, nf1 = _process_a(NCHUNK_A - 1, chunk_s, chunk_d, carry)

    @pl.when(cur0 > 0)
    def _():
        _flush_a0(nf0)

    @pl.when(cur1 > 0)
    def _():
        _flush_a1(nf1)

    cntbuf[...] = lax.broadcast(nf0 * CH + cur0, (16,))
    pltpu.sync_copy(cntbuf, sp_cnt.at[pl.ds(s * 32, 16)])
    cntbuf[...] = lax.broadcast(nf1 * CH + cur1, (16,))
    pltpu.sync_copy(cntbuf, sp_cnt.at[pl.ds(s * 32 + 16, 16)])

    plsc.subcore_barrier()

    pltpu.make_async_copy(z_acc, acc.at[pl.ds(0, ROWS * D)], sem_z0).wait()
    pltpu.make_async_copy(z_deg, dacc, sem_z1).wait()
    pltpu.sync_copy(sp_cnt, cntv)
    offs = [iota + g * 16 for g in range(16)]

    def _flush_b(limit):
        for q in range(GB // 16):
            ok = q * 16 + iota < limit
            v16 = st_a[pl.ds(q * 16, 16)]
            st_a[pl.ds(q * 16, 16)] = jnp.where(ok, v16, 0)
            b16 = st_b[pl.ds(q * 16, 16)]
            st_b[pl.ds(q * 16, 16)] = jnp.where(ok, b16, ROWS * D)
        pltpu.sync_copy(x_hbm.at[st_a.at[pl.ds(0, GB)]], hbuf)

        @plsc.parallel_loop(0, GB, 1, unroll=4)
        def _edge(i):
            b16 = plsc.load_gather(st_b, [lax.broadcast(i, (16,))])
            for g in range(16):
                vals = hbuf[i, pl.ds(g * 16, 16)]
                plsc.addupdate_scatter(acc, [b16 + offs[g]], vals)

    def _region_b(p, cur):
        c16 = cntv[pl.ds((p * 2 + c) * 16, 16)]
        cnt_p = jnp.max(c16)
        nchunk = (cnt_p + CH - 1) // CH

        def _chunk_b(t, cur):
            off = p * REGION + t * CH + c * (REGION - 2 * t * CH - CH)
            pltpu.sync_copy(sp_src.at[pl.ds(off, CH)], chunk_s)
            pltpu.sync_copy(sp_dst.at[pl.ds(off, CH)], chunk_d)

            def _vec_b(v, cur):
                s16 = chunk_s[pl.ds(v * 16, 16)]
                d16 = chunk_d[pl.ds(v * 16, 16)]
                pos = t * CH + v * 16 + iota
                keep = (pos < cnt_p) & (d16 >= lo) & (d16 < lo + ROWS)
                l16 = jnp.where(keep, d16 - lo, 0)
                plsc.addupdate_scatter(dacc, [l16 * 16 + iota], ones_f,
                                       mask=keep)
                plsc.store_compressed(st_a.at[pl.ds(cur, 16)], s16, mask=keep)
                plsc.store_compressed(st_b.at[pl.ds(cur, 16)], l16 * D,
                                      mask=keep)
                cur = cur + jnp.max(plsc.all_reduce_population_count(keep))
                do_flush = cur >= GB

                @pl.when(do_flush)
                def _():
                    _flush_b(GB)
                    st_a[pl.ds(0, 16)] = st_a[pl.ds(GB, 16)]
                    st_b[pl.ds(0, 16)] = st_b[pl.ds(GB, 16)]

                return jnp.where(do_flush, cur - GB, cur)

            return lax.fori_loop(0, VECS, _vec_b, cur)

        return lax.fori_loop(0, nchunk, _chunk_b, cur)

    cur = lax.fori_loop(0, NSUB, _region_b, jnp.int32(0))

    @pl.when(cur > 0)
    def _():
        _flush_b(cur)

    pltpu.sync_copy(acc.at[pl.ds(0, ROWS * D)],
                    s_out.at[pl.ds(w * (ROWS * D), ROWS * D)])
    pltpu.sync_copy(dacc, deg_out.at[pl.ds(w * (ROWS * 16), ROWS * 16)])


_sc_cp = pltpu.CompilerParams()
if "needs_layout_passes" in pltpu.CompilerParams.__dataclass_fields__:
    _sc_cp = dataclasses.replace(_sc_cp, needs_layout_passes=False)

_sc_agg = pl.kernel(
    _sc_body,
    compiler_params=_sc_cp,
    out_type=(
        jax.ShapeDtypeStruct((NPAD * D,), _f32),
        jax.ShapeDtypeStruct((NPAD * 16,), _f32),
    ),
    mesh=plsc.VectorSubcoreMesh(core_axis_name="c", subcore_axis_name="s"),
    scratch_types=[
        pltpu.VMEM((N,), _i32),
        pltpu.VMEM((CH,), _i32),
        pltpu.VMEM((CH,), _i32),
        pltpu.VMEM((CH,), _i32),
        pltpu.VMEM((CH,), _i32),
        pltpu.VMEM((STAGE,), _i32),
        pltpu.VMEM((STAGE,), _i32),
        pltpu.VMEM((STAGE,), _i32),
        pltpu.VMEM((STAGE,), _i32),
        pltpu.VMEM((16,), _i32),
        pltpu.VMEM((NSUB * 32,), _i32),
        pltpu.VMEM((GB, D), _f32),
        pltpu.VMEM((ROWS * D + 256,), _f32),
        pltpu.VMEM((ROWS * 16,), _f32),
        pltpu.SemaphoreType.DMA,
        pltpu.SemaphoreType.DMA,
        pltpu.SemaphoreType.DMA,
        pltpu.SemaphoreType.DMA,
        pltpu.SemaphoreType.DMA,
        pltpu.SemaphoreType.DMA,
        pltpu.VMEM_SHARED((NSUB * REGION,), _i32),
        pltpu.VMEM_SHARED((NSUB * REGION,), _i32),
        pltpu.VMEM_SHARED((NSUB * 32,), _i32),
    ],
)


def _tc_body(s_ref, deg_ref, w_ref, b_ref, a_ref, o_ref):
    sblk = s_ref[...]
    m = lax.dot_general(sblk, w_ref[...], (((1,), (1,)), ((), ())),
                        preferred_element_type=_f32,
                        precision=lax.Precision.HIGHEST)
    deg0 = jnp.sum(deg_ref[...], axis=1, keepdims=True)
    pos = deg0 > 0
    inv = jnp.where(pos, 1.0 / deg0, 0.0)
    a = a_ref[0, 0]
    o_ref[...] = a * inv * m + jnp.where(pos, a, 0.0) * b_ref[...]


def _tc_finish(S, deg, W, b2, a2):
    blk = 1000
    return pl.pallas_call(
        _tc_body,
        grid=(N // blk,),
        in_specs=[
            pl.BlockSpec((blk, D), lambda i: (i, 0)),
            pl.BlockSpec((blk, 16), lambda i: (i, 0)),
            pl.BlockSpec((D, D), lambda i: (0, 0)),
            pl.BlockSpec((1, D), lambda i: (0, 0)),
            pl.BlockSpec(memory_space=pltpu.SMEM),
        ],
        out_specs=pl.BlockSpec((blk, D), lambda i: (i, 0)),
        out_shape=jax.ShapeDtypeStruct((N, D), _f32),
    )(S, deg, W, b2, a2)


def kernel(x, edge_index, batch_index, node_rankings, W, b, alpha):
    src = edge_index[0]
    dst = edge_index[1]
    rank = node_rankings[0]
    z_acc = jnp.zeros((ROWS * D,), _f32)
    z_deg = jnp.zeros((ROWS * 16,), _f32)
    s_flat, d_flat = _sc_agg(x, src, dst, rank, z_acc, z_deg)
    S = s_flat.reshape(NPAD, D)
    deg = d_flat.reshape(NPAD, 16)
    return _tc_finish(S, deg, W, b.reshape(1, D),
                      alpha.reshape(1, 1).astype(_f32))


# speedup vs baseline: 1.0274x; 1.0274x over previous
import dataclasses
import functools

import jax
import jax.numpy as jnp
from jax import lax
from jax.experimental import pallas as pl
from jax.experimental.pallas import tpu as pltpu
from jax.experimental.pallas import tpu_sc as plsc

N = 10000
D = 256
E = 160000
K_ACTIVE = 1000
NSUB = 16
NW = 32
ROWS = 320
NPAD = NW * ROWS

EDGES_PER_SCAN = E // NSUB
CH = 400
NCHUNK_A = EDGES_PER_SCAN // CH
VECS = CH // 16
REGION = 10400
HALF_N = 16 * ROWS
GB = 16
STAGE = CH + 16

_i32 = jnp.int32
_f32 = jnp.float32


def _sc_body(x_hbm, src_hbm, dst_hbm, rank_hbm, z_acc, z_deg, alpha_hbm,
             s_out,
             rank_v, chunk_s, chunk_d, chunk_s2, chunk_d2,
             st_a, st_b, st_a2, st_b2,
             cntbuf, cntv, hbuf, acc, dacc, alpha_v,
             sem_s0, sem_d0, sem_s1, sem_d1, sem_z0, sem_z1,
             sp_src, sp_dst, sp_cnt):
    c = lax.axis_index("c")
    s = lax.axis_index("s")
    w = c * NSUB + s
    lo = w * ROWS
    iota = lax.iota(_i32, 16)
    ones_f = jnp.ones((16,), _f32)

    pltpu.async_copy(z_acc, acc.at[pl.ds(0, ROWS * D)], sem_z0)
    pltpu.async_copy(z_deg, dacc, sem_z1)

    pltpu.sync_copy(rank_hbm, rank_v)
    pltpu.sync_copy(alpha_hbm, alpha_v)
    base = s * EDGES_PER_SCAN

    def _flush_a0(nf):
        pltpu.sync_copy(st_a.at[pl.ds(0, CH)],
                        sp_src.at[pl.ds(s * REGION + nf * CH, CH)])
        pltpu.sync_copy(st_b.at[pl.ds(0, CH)],
                        sp_dst.at[pl.ds(s * REGION + nf * CH, CH)])

    def _flush_a1(nf):
        off = s * REGION + REGION - (nf + 1) * CH
        pltpu.sync_copy(st_a2.at[pl.ds(0, CH)],
                        sp_src.at[pl.ds(off, CH)])
        pltpu.sync_copy(st_b2.at[pl.ds(0, CH)],
                        sp_dst.at[pl.ds(off, CH)])

    def _start_a(t, cs, cd, ss, sd):
        pltpu.async_copy(src_hbm.at[pl.ds(base + t * CH, CH)], cs, ss)
        pltpu.async_copy(dst_hbm.at[pl.ds(base + t * CH, CH)], cd, sd)

    def _wait_a(t, cs, cd, ss, sd):
        pltpu.make_async_copy(src_hbm.at[pl.ds(base + t * CH, CH)],
                              cs, ss).wait()
        pltpu.make_async_copy(dst_hbm.at[pl.ds(base + t * CH, CH)],
                              cd, sd).wait()

    def _process_a(t, cs, cd, carry):
        def _vec_a(v, carry):
            cur0, nf0, cur1, nf1 = carry
            s16 = cs[pl.ds(v * 16, 16)]
            d16 = cd[pl.ds(v * 16, 16)]
            r16 = plsc.load_gather(rank_v, [s16])
            act = r16 <= K_ACTIVE
            keep0 = act & (d16 < HALF_N)
            keep1 = act & (d16 >= HALF_N)
            plsc.store_compressed(st_a.at[pl.ds(cur0, 16)], s16, mask=keep0)
            plsc.store_compressed(st_b.at[pl.ds(cur0, 16)], d16, mask=keep0)
            plsc.store_compressed(st_a2.at[pl.ds(cur1, 16)], s16, mask=keep1)
            plsc.store_compressed(st_b2.at[pl.ds(cur1, 16)], d16, mask=keep1)
            cur0 = cur0 + jnp.max(plsc.all_reduce_population_count(keep0))
            cur1 = cur1 + jnp.max(plsc.all_reduce_population_count(keep1))
            f0 = cur0 >= CH
            f1 = cur1 >= CH

            @pl.when(f0)
            def _():
                _flush_a0(nf0)
                st_a[pl.ds(0, 16)] = st_a[pl.ds(CH, 16)]
                st_b[pl.ds(0, 16)] = st_b[pl.ds(CH, 16)]

            @pl.when(f1)
            def _():
                _flush_a1(nf1)
                st_a2[pl.ds(0, 16)] = st_a2[pl.ds(CH, 16)]
                st_b2[pl.ds(0, 16)] = st_b2[pl.ds(CH, 16)]

            cur0 = jnp.where(f0, cur0 - CH, cur0)
            nf0 = jnp.where(f0, nf0 + 1, nf0)
            cur1 = jnp.where(f1, cur1 - CH, cur1)
            nf1 = jnp.where(f1, nf1 + 1, nf1)
            return cur0, nf0, cur1, nf1

        return lax.fori_loop(0, VECS, _vec_a, carry)

    _start_a(0, chunk_s, chunk_d, sem_s0, sem_d0)

    def _pair_a(i, carry):
        t0 = 2 * i
        _start_a(t0 + 1, chunk_s2, chunk_d2, sem_s1, sem_d1)
        _wait_a(t0, chunk_s, chunk_d, sem_s0, sem_d0)
        carry = _process_a(t0, chunk_s, chunk_d, carry)
        _start_a(t0 + 2, chunk_s, chunk_d, sem_s0, sem_d0)
        _wait_a(t0 + 1, chunk_s2, chunk_d2, sem_s1, sem_d1)
        return _process_a(t0 + 1, chunk_s2, chunk_d2, carry)

    zero4 = (jnp.int32(0),) * 4
    carry = lax.fori_loop(0, (NCHUNK_A - 1) // 2, _pair_a, zero4)
    _wait_a(NCHUNK_A - 1, chunk_s, chunk_d, sem_s0, sem_d0)
    cur0, nf0, cur1, nf1 = _process_a(NCHUNK_A - 1, chunk_s, chunk_d, carry)

    @pl.when(cur0 > 0)
    def _():
        _flush_a0(nf0)

    @pl.when(cur1 > 0)
    def _():
        _flush_a1(nf1)

    cntbuf[...] = lax.broadcast(nf0 * CH + cur0, (16,))
    pltpu.sync_copy(cntbuf, sp_cnt.at[pl.ds(s * 32, 16)])
    cntbuf[...] = lax.broadcast(nf1 * CH + cur1, (16,))
    pltpu.sync_copy(cntbuf, sp_cnt.at[pl.ds(s * 32 + 16, 16)])

    plsc.subcore_barrier()

    pltpu.make_async_copy(z_acc, acc.at[pl.ds(0, ROWS * D)], sem_z0).wait()
    pltpu.make_async_copy(z_deg, dacc, sem_z1).wait()
    pltpu.sync_copy(sp_cnt, cntv)
    offs = [iota + g * 16 for g in range(16)]

    def _flush_b(limit):
        for q in range(GB // 16):
            ok = q * 16 + iota < limit
            v16 = st_a[pl.ds(q * 16, 16)]
            st_a[pl.ds(q * 16, 16)] = jnp.where(ok, v16, 0)
            b16 = st_b[pl.ds(q * 16, 16)]
            st_b[pl.ds(q * 16, 16)] = jnp.where(ok, b16, ROWS * D)
        pltpu.sync_copy(x_hbm.at[st_a.at[pl.ds(0, GB)]], hbuf)

        @plsc.parallel_loop(0, GB, 1, unroll=4)
        def _edge(i):
            b16 = plsc.load_gather(st_b, [lax.broadcast(i, (16,))])
            for g in range(16):
                vals = hbuf[i, pl.ds(g * 16, 16)]
                plsc.addupdate_scatter(acc, [b16 + offs[g]], vals)

    def _region_b(p, cur):
        c16 = cntv[pl.ds((p * 2 + c) * 16, 16)]
        cnt_p = jnp.max(c16)
        nchunk = (cnt_p + CH - 1) // CH

        def _chunk_b(t, cur):
            off = p * REGION + t * CH + c * (REGION - 2 * t * CH - CH)
            pltpu.sync_copy(sp_src.at[pl.ds(off, CH)], chunk_s)
            pltpu.sync_copy(sp_dst.at[pl.ds(off, CH)], chunk_d)

            def _vec_b(v, cur):
                s16 = chunk_s[pl.ds(v * 16, 16)]
                d16 = chunk_d[pl.ds(v * 16, 16)]
                pos = t * CH + v * 16 + iota
                keep = (pos < cnt_p) & (d16 >= lo) & (d16 < lo + ROWS)
                l16 = jnp.where(keep, d16 - lo, 0)
                plsc.addupdate_scatter(dacc, [l16 * 16 + iota], ones_f,
                                       mask=keep)
                plsc.store_compressed(st_a.at[pl.ds(cur, 16)], s16, mask=keep)
                plsc.store_compressed(st_b.at[pl.ds(cur, 16)], l16 * D,
                                      mask=keep)
                cur = cur + jnp.max(plsc.all_reduce_population_count(keep))
                do_flush = cur >= GB

                @pl.when(do_flush)
                def _():
                    _flush_b(GB)
                    st_a[pl.ds(0, 16)] = st_a[pl.ds(GB, 16)]
                    st_b[pl.ds(0, 16)] = st_b[pl.ds(GB, 16)]

                return jnp.where(do_flush, cur - GB, cur)

            return lax.fori_loop(0, VECS, _vec_b, cur)

        return lax.fori_loop(0, nchunk, _chunk_b, cur)

    cur = lax.fori_loop(0, NSUB, _region_b, jnp.int32(0))

    @pl.when(cur > 0)
    def _():
        _flush_b(cur)

    a16 = alpha_v[...]

    def _scale(r, _):
        d16 = dacc[pl.ds(r * 16, 16)]
        deg16 = lax.broadcast(jnp.sum(d16), (16,))
        m16 = jnp.where(deg16 > 0, a16 / deg16, 0.0)
        for g in range(16):
            sl = pl.ds(r * D + g * 16, 16)
            acc[sl] = acc[sl] * m16
        return 0

    lax.fori_loop(0, ROWS, _scale, 0)

    @pl.when(w < NW - 1)
    def _():
        pltpu.sync_copy(acc.at[pl.ds(0, ROWS * D)],
                        s_out.at[pl.ds(w * (ROWS * D), ROWS * D)])

    @pl.when(w == NW - 1)
    def _():
        pltpu.sync_copy(acc.at[pl.ds(0, (N - (NW - 1) * ROWS) * D)],
                        s_out.at[pl.ds(w * (ROWS * D),
                                       (N - (NW - 1) * ROWS) * D)])


_sc_cp = pltpu.CompilerParams()
if "needs_layout_passes" in pltpu.CompilerParams.__dataclass_fields__:
    _sc_cp = dataclasses.replace(_sc_cp, needs_layout_passes=False)

_sc_agg = pl.kernel(
    _sc_body,
    compiler_params=_sc_cp,
    out_type=(jax.ShapeDtypeStruct((N * D,), _f32),),
    mesh=plsc.VectorSubcoreMesh(core_axis_name="c", subcore_axis_name="s"),
    scratch_types=[
        pltpu.VMEM((N,), _i32),
        pltpu.VMEM((CH,), _i32),
        pltpu.VMEM((CH,), _i32),
        pltpu.VMEM((CH,), _i32),
        pltpu.VMEM((CH,), _i32),
        pltpu.VMEM((STAGE,), _i32),
        pltpu.VMEM((STAGE,), _i32),
        pltpu.VMEM((STAGE,), _i32),
        pltpu.VMEM((STAGE,), _i32),
        pltpu.VMEM((16,), _i32),
        pltpu.VMEM((NSUB * 32,), _i32),
        pltpu.VMEM((GB, D), _f32),
        pltpu.VMEM((ROWS * D + 256,), _f32),
        pltpu.VMEM((ROWS * 16,), _f32),
        pltpu.VMEM((16,), _f32),
        pltpu.SemaphoreType.DMA,
        pltpu.SemaphoreType.DMA,
        pltpu.SemaphoreType.DMA,
        pltpu.SemaphoreType.DMA,
        pltpu.SemaphoreType.DMA,
        pltpu.SemaphoreType.DMA,
        pltpu.VMEM_SHARED((NSUB * REGION,), _i32),
        pltpu.VMEM_SHARED((NSUB * REGION,), _i32),
        pltpu.VMEM_SHARED((NSUB * 32,), _i32),
    ],
)


def _tc_body(x_ref, w_ref, b_ref, o_ref):
    o_ref[...] = lax.dot_general(x_ref[...], w_ref[...],
                                 (((1,), (1,)), ((), ())),
                                 preferred_element_type=_f32,
                                 precision=lax.Precision.HIGHEST) + b_ref[...]


def _tc_linear(x, W, b2):
    blk = 1000
    return pl.pallas_call(
        _tc_body,
        grid=(N // blk,),
        in_specs=[
            pl.BlockSpec((blk, D), lambda i: (i, 0)),
            pl.BlockSpec((D, D), lambda i: (0, 0)),
            pl.BlockSpec((1, D), lambda i: (0, 0)),
        ],
        out_specs=pl.BlockSpec((blk, D), lambda i: (i, 0)),
        out_shape=jax.ShapeDtypeStruct((N, D), _f32),
    )(x, W, b2)


def kernel(x, edge_index, batch_index, node_rankings, W, b, alpha):
    src = edge_index[0]
    dst = edge_index[1]
    rank = node_rankings[0]
    h = _tc_linear(x, W, b.reshape(1, D))
    z_acc = jnp.zeros((ROWS * D,), _f32)
    z_deg = jnp.zeros((ROWS * 16,), _f32)
    a16 = jnp.broadcast_to(alpha.astype(_f32), (16,))
    (out_flat,) = _sc_agg(h, src, dst, rank, z_acc, z_deg, a16)
    return out_flat.reshape(N, D)
